# baseline (speedup 1.0000x reference)
import jax
import jax.numpy as jnp
from jax.experimental import pallas as pl

B, S, D = 4, 2048, 1024
S_BLK = 256


def _add_body(x_ref, p_ref, o_ref):
    o_ref[...] = x_ref[...] + p_ref[...][None, :, :]


def kernel(inputs, pos_weight):
    return pl.pallas_call(
        _add_body,
        grid=(B, S // S_BLK),
        in_specs=[
            pl.BlockSpec((1, S_BLK, D), lambda b, j: (b, j, 0)),
            pl.BlockSpec((S_BLK, 1), lambda b, j: (j, 0)),
        ],
        out_specs=pl.BlockSpec((1, S_BLK, D), lambda b, j: (b, j, 0)),
        out_shape=jax.ShapeDtypeStruct((B, S, D), jnp.float32),
    )(inputs, pos_weight)


# speedup vs baseline: 1.4394x; 1.4394x over previous
import jax
import jax.numpy as jnp
from jax.experimental import pallas as pl

B, S, D = 4, 2048, 1024
S_BLK = 1024


def _add_body(x_ref, p_ref, o_ref):
    o_ref[...] = x_ref[...] + p_ref[...][None, :, :]


def kernel(inputs, pos_weight):
    return pl.pallas_call(
        _add_body,
        grid=(B, S // S_BLK),
        in_specs=[
            pl.BlockSpec((1, S_BLK, D), lambda b, j: (b, j, 0)),
            pl.BlockSpec((S_BLK, 1), lambda b, j: (j, 0)),
        ],
        out_specs=pl.BlockSpec((1, S_BLK, D), lambda b, j: (b, j, 0)),
        out_shape=jax.ShapeDtypeStruct((B, S, D), jnp.float32),
    )(inputs, pos_weight)


# speedup vs baseline: 1.5935x; 1.1071x over previous
import jax
import jax.numpy as jnp
from jax.experimental import pallas as pl

B, S, D = 4, 2048, 1024
S_BLK = 1024


def _add_body(x_ref, p_ref, o_ref):
    o_ref[...] = x_ref[...] + p_ref[...][None, :, :]


def kernel(inputs, pos_weight):
    return pl.pallas_call(
        _add_body,
        grid=(B,),
        in_specs=[
            pl.BlockSpec((1, S, D), lambda b: (b, 0, 0)),
            pl.BlockSpec((S, 1), lambda b: (0, 0)),
        ],
        out_specs=pl.BlockSpec((1, S, D), lambda b: (b, 0, 0)),
        out_shape=jax.ShapeDtypeStruct((B, S, D), jnp.float32),
    )(inputs, pos_weight)
